# Initial kernel scaffold; baseline (speedup 1.0000x reference)
import jax
import jax.numpy as jnp
from jax.experimental import pallas as pl


def kernel(z, W, training):
    raise NotImplementedError("write your pallas kernel here")


# speedup vs baseline: 51.3606x; 51.3606x over previous
import functools

import jax
import jax.numpy as jnp
from jax.experimental import pallas as pl
from jax.experimental.pallas import tpu as pltpu
from jax.experimental.pallas import tpu_sc as plsc

NUM_EMBED = 1024
EMBED_DIM = 256
BETA = 0.25

N_ROWS = 16 * 32 * 32
BLK = 2048
N_STEPS = N_ROWS // BLK


def _vq_tc_body(z_ref, w_ref, enc_ref, idx_ref, zq_ref, loss_ref, perp_ref,
                cnt_ref, acc_ref):
    step = pl.program_id(0)

    @pl.when(step == 0)
    def _():
        cnt_ref[...] = jnp.zeros_like(cnt_ref)
        acc_ref[0] = 0.0

    zb = z_ref[...]
    w = w_ref[...]

    wsq = jnp.sum(w * w, axis=1, keepdims=True)
    wn = w / jnp.maximum(jnp.sqrt(wsq), 1e-12)
    zsq = jnp.sum(zb * zb, axis=1, keepdims=True)
    nz = zb / jnp.maximum(jnp.sqrt(zsq), 1e-12)

    d = jax.lax.dot_general(nz, wn, (((1,), (1,)), ((), ())),
                            preferred_element_type=jnp.float32)

    dmax = jnp.max(d, axis=1, keepdims=True)
    iota = jax.lax.broadcasted_iota(jnp.int32, d.shape, 1)
    idx = jnp.max(jnp.where(d == dmax, iota, -1), axis=1, keepdims=True)

    enc = (iota == idx).astype(jnp.float32)
    enc_ref[...] = enc
    idx_ref[...] = idx
    cnt_ref[...] += jnp.sum(enc, axis=0, keepdims=True)

    zq = jax.lax.dot_general(enc, w, (((1,), (0,)), ((), ())),
                             preferred_element_type=jnp.float32)
    zq_ref[...] = zq

    diff = zq - zb
    bsum = jnp.sum(diff * diff)
    total = acc_ref[0] + bsum
    acc_ref[0] = total

    @pl.when(step == N_STEPS - 1)
    def _():
        loss_ref[0] = (1.0 + BETA) * total / (N_ROWS * EMBED_DIM)
        p = cnt_ref[...] / N_ROWS
        perp_ref[0] = jnp.exp(-jnp.sum(p * jnp.log(p + 1e-10)))


@jax.jit
def _vq_tc(z_flat, w):
    out_shapes = (
        jax.ShapeDtypeStruct((N_ROWS, NUM_EMBED), jnp.float32),
        jax.ShapeDtypeStruct((N_ROWS, 1), jnp.int32),
        jax.ShapeDtypeStruct((N_ROWS, EMBED_DIM), jnp.float32),
        jax.ShapeDtypeStruct((1,), jnp.float32),
        jax.ShapeDtypeStruct((1,), jnp.float32),
    )
    return pl.pallas_call(
        _vq_tc_body,
        grid=(N_STEPS,),
        in_specs=[
            pl.BlockSpec((BLK, EMBED_DIM), lambda i: (i, 0)),
            pl.BlockSpec((NUM_EMBED, EMBED_DIM), lambda i: (0, 0)),
        ],
        out_specs=(
            pl.BlockSpec((BLK, NUM_EMBED), lambda i: (i, 0)),
            pl.BlockSpec((BLK, 1), lambda i: (i, 0)),
            pl.BlockSpec((BLK, EMBED_DIM), lambda i: (i, 0)),
            pl.BlockSpec(memory_space=pltpu.SMEM),
            pl.BlockSpec(memory_space=pltpu.SMEM),
        ),
        out_shape=out_shapes,
        scratch_shapes=[
            pltpu.VMEM((1, NUM_EMBED), jnp.float32),
            pltpu.SMEM((1,), jnp.float32),
        ],
    )(z_flat, w)


def kernel(z, W, training):
    zp = jnp.transpose(z, (0, 2, 3, 1))
    z_flat = zp.reshape(N_ROWS, EMBED_DIM)

    enc, idx2d, zq, loss, perp = _vq_tc(z_flat, W)

    z_q_out = jnp.transpose(zq.reshape(16, 32, 32, EMBED_DIM), (0, 3, 1, 2))
    encoding_indices = idx2d.reshape(N_ROWS)
    return (loss[0], z_q_out, perp[0], enc, encoding_indices)
